# baseline (speedup 1.0000x reference)
import functools

import jax
import jax.numpy as jnp
from jax import lax
from jax.experimental import pallas as pl
from jax.experimental.pallas import tpu as pltpu
from jax.experimental.pallas import tpu_sc as plsc

P = 0.2
B, C, D = 4, 4096, 2048
DROP_NUM = int(round(P * C))
SCALE = 1.0 / (1.0 - P)
CBLK = 512
NV = C // 16


def _lane_sum(acc):
    total = jnp.int32(0)
    for j in range(16):
        total = total + acc[j]
    return total


def _count_le(bits_v, mid):
    def body(i, acc):
        v = bits_v[pl.ds(i * 16, 16)]
        return acc + jnp.where(v <= mid, 1, 0).astype(jnp.int32)
    return _lane_sum(lax.fori_loop(0, NV, body, jnp.zeros((16,), jnp.int32)))


def _sc_select_body(bits_hbm, scale_hbm, bits_v, out_v):
    wid = lax.axis_index("s") * 2 + lax.axis_index("c")

    @pl.when(wid < B)
    def _():
        pltpu.sync_copy(bits_hbm.at[wid], bits_v)

        def sbody(_, carry):
            lo, hi = carry
            mid = lo + (hi - lo) // 2
            ge = _count_le(bits_v, mid) >= DROP_NUM
            return (jnp.where(ge, lo, mid + 1), jnp.where(ge, mid, hi))
        lo, hi = lax.fori_loop(0, 31, sbody,
                               (jnp.int32(0), jnp.int32(0x3F800000)))
        tbits = lo

        n_less = _count_le(bits_v, tbits - 1)
        need = DROP_NUM - n_less

        def ibody(_, carry):
            ilo, ihi = carry
            mid = ilo + (ihi - ilo + 1) // 2

            def cbody(i, acc):
                v = bits_v[pl.ds(i * 16, 16)]
                idx = lax.iota(jnp.int32, 16) + i * 16
                hit = (v == tbits) & (idx <= mid)
                return acc + jnp.where(hit, 1, 0).astype(jnp.int32)
            cnt = _lane_sum(lax.fori_loop(0, NV, cbody,
                                          jnp.zeros((16,), jnp.int32)))
            ge = cnt >= need
            return (jnp.where(ge, ilo, mid), jnp.where(ge, mid, ihi))
        ilo, ihi = lax.fori_loop(0, 13, ibody,
                                 (jnp.int32(-1), jnp.int32(C - 1)))
        itop = ihi

        def mbody(i, _):
            v = bits_v[pl.ds(i * 16, 16)]
            idx = lax.iota(jnp.int32, 16) + i * 16
            drop = (v < tbits) | ((v == tbits) & (idx <= itop))
            out_v[pl.ds(i * 16, 16)] = jnp.where(drop, 0.0, SCALE).astype(
                jnp.float32)
            return 0
        lax.fori_loop(0, NV, mbody, 0)

        pltpu.sync_copy(out_v, scale_hbm.at[wid])


@functools.cache
def _sc_select():
    mesh = plsc.VectorSubcoreMesh(core_axis_name="c", subcore_axis_name="s")
    return pl.kernel(
        _sc_select_body,
        out_type=jax.ShapeDtypeStruct((B, C), jnp.float32),
        mesh=mesh,
        scratch_types=[
            pltpu.VMEM((C,), jnp.int32),
            pltpu.VMEM((C,), jnp.float32),
        ],
    )


def _mul_kernel(x_ref, scale_ref, o_ref):
    c = pl.program_id(1)
    s = scale_ref[0, 0, pl.ds(c * CBLK, CBLK)]
    o_ref[0] = x_ref[0] * s[:, None]


@jax.jit
def kernel(inputs):
    rand = jax.random.uniform(jax.random.key(42), (B, C), dtype=jnp.float32)
    bits = jax.lax.bitcast_convert_type(rand, jnp.int32)
    scale = _sc_select()(bits)
    out = pl.pallas_call(
        _mul_kernel,
        grid=(B, C // CBLK),
        in_specs=[
            pl.BlockSpec((1, CBLK, D), lambda b, c: (b, c, 0)),
            pl.BlockSpec((1, 1, C), lambda b, c: (b, 0, 0)),
        ],
        out_specs=pl.BlockSpec((1, CBLK, D), lambda b, c: (b, c, 0)),
        out_shape=jax.ShapeDtypeStruct((B, C, D), jnp.float32),
    )(inputs, scale.reshape(B, 1, C))
    return out


# speedup vs baseline: 1.3580x; 1.3580x over previous
import functools

import jax
import jax.numpy as jnp
from jax import lax
from jax.experimental import pallas as pl
from jax.experimental.pallas import tpu as pltpu
from jax.experimental.pallas import tpu_sc as plsc

P = 0.2
B, C, D = 4, 4096, 2048
DROP_NUM = int(round(P * C))
SCALE = 1.0 / (1.0 - P)
CBLK = 512
NV = C // 16
UNROLL = 8
NB = NV // UNROLL


def _lane_sum(acc):
    total = jnp.int32(0)
    for j in range(16):
        total = total + acc[j]
    return total


def _count_le(bits_v, mid):
    def body(i, acc):
        base = i * (16 * UNROLL)
        for u in range(UNROLL):
            v = bits_v[pl.ds(base + u * 16, 16)]
            acc = acc + jnp.where(v <= mid, 1, 0).astype(jnp.int32)
        return acc
    return _lane_sum(lax.fori_loop(0, NB, body, jnp.zeros((16,), jnp.int32)))


def _sc_select_body(bits_hbm, scale_hbm, bits_v, out_v):
    wid = lax.axis_index("s") * 2 + lax.axis_index("c")

    @pl.when(wid < B)
    def _():
        pltpu.sync_copy(bits_hbm.at[wid], bits_v)

        def sbody(_, carry):
            lo, hi = carry
            mid = lo + (hi - lo) // 2
            ge = _count_le(bits_v, mid) >= DROP_NUM
            return (jnp.where(ge, lo, mid + 1), jnp.where(ge, mid, hi))
        lo, hi = lax.fori_loop(0, 31, sbody,
                               (jnp.int32(0), jnp.int32(0x3F800000)))
        tbits = lo

        n_at = _count_le(bits_v, tbits)

        def tie_path():
            n_less = _count_le(bits_v, tbits - 1)
            need = DROP_NUM - n_less

            def ibody(_, carry):
                ilo, ihi = carry
                mid = ilo + (ihi - ilo + 1) // 2

                def cbody(i, acc):
                    base = i * (16 * UNROLL)
                    for u in range(UNROLL):
                        v = bits_v[pl.ds(base + u * 16, 16)]
                        idx = lax.iota(jnp.int32, 16) + (base + u * 16)
                        hit = (v == tbits) & (idx <= mid)
                        acc = acc + jnp.where(hit, 1, 0).astype(jnp.int32)
                    return acc
                cnt = _lane_sum(lax.fori_loop(0, NB, cbody,
                                              jnp.zeros((16,), jnp.int32)))
                ge = cnt >= need
                return (jnp.where(ge, ilo, mid), jnp.where(ge, mid, ihi))
            _, ihi = lax.fori_loop(0, 13, ibody,
                                   (jnp.int32(-1), jnp.int32(C - 1)))
            return ihi

        itop = lax.cond(n_at == DROP_NUM, lambda: jnp.int32(C - 1), tie_path)

        def mbody(i, _):
            base = i * (16 * UNROLL)
            for u in range(UNROLL):
                v = bits_v[pl.ds(base + u * 16, 16)]
                idx = lax.iota(jnp.int32, 16) + (base + u * 16)
                drop = (v < tbits) | ((v == tbits) & (idx <= itop))
                out_v[pl.ds(base + u * 16, 16)] = jnp.where(
                    drop, 0.0, SCALE).astype(jnp.float32)
            return 0
        lax.fori_loop(0, NB, mbody, 0)

        pltpu.sync_copy(out_v, scale_hbm.at[wid, 0])


@functools.cache
def _sc_select():
    mesh = plsc.VectorSubcoreMesh(core_axis_name="c", subcore_axis_name="s")
    return pl.kernel(
        _sc_select_body,
        out_type=jax.ShapeDtypeStruct((B, 1, C), jnp.float32),
        mesh=mesh,
        scratch_types=[
            pltpu.VMEM((C,), jnp.int32),
            pltpu.VMEM((C,), jnp.float32),
        ],
    )


def _mul_kernel(x_ref, scale_ref, o_ref):
    c = pl.program_id(1)
    s = scale_ref[0, 0, pl.ds(c * CBLK, CBLK)]
    o_ref[0] = x_ref[0] * s[:, None]


@jax.jit
def kernel(inputs):
    rand = jax.random.uniform(jax.random.key(42), (B, C), dtype=jnp.float32)
    bits = jax.lax.bitcast_convert_type(rand, jnp.int32)
    scale = _sc_select()(bits)
    out = pl.pallas_call(
        _mul_kernel,
        grid=(B, C // CBLK),
        in_specs=[
            pl.BlockSpec((1, CBLK, D), lambda b, c: (b, c, 0)),
            pl.BlockSpec((1, 1, C), lambda b, c: (b, 0, 0)),
        ],
        out_specs=pl.BlockSpec((1, CBLK, D), lambda b, c: (b, c, 0)),
        out_shape=jax.ShapeDtypeStruct((B, C, D), jnp.float32),
    )(inputs, scale)
    return out
